# baseline (speedup 1.0000x reference)
import functools

import jax
import jax.numpy as jnp
from jax import lax
from jax.experimental import pallas as pl
from jax.experimental.pallas import tpu as pltpu
from jax.experimental.pallas import tpu_sc as plsc

NB = 8192
D = 64
TM = 1024

def _argmin_body(x_ref, cb_ref, xsq_ref, csq_ref, iota_ref, idx_ref):
    xs = x_ref[...] * -2.0
    mm = lax.dot_general(xs, cb_ref[...], (((1,), (1,)), ((), ())),
                         preferred_element_type=jnp.float32)
    dist = (xsq_ref[...] + mm) + csq_ref[...]
    m = jnp.min(dist, axis=-1, keepdims=True)
    cand = jnp.where(dist == m, iota_ref[...], jnp.float32(NB))
    col = jnp.min(cand, axis=-1, keepdims=True).astype(jnp.int32)
    idx_ref[...] = lax.transpose(col, (1, 0))


def _argmin_call(x, cb, xsq, csq):
    M = x.shape[0]
    iota_row = jnp.arange(NB, dtype=jnp.float32)[None, :]
    return pl.pallas_call(
        _argmin_body,
        grid=(M // TM,),
        in_specs=[
            pl.BlockSpec((TM, D), lambda i: (i, 0)),
            pl.BlockSpec((NB, D), lambda i: (0, 0)),
            pl.BlockSpec((TM, 1), lambda i: (i, 0)),
            pl.BlockSpec((1, NB), lambda i: (0, 0)),
            pl.BlockSpec((1, NB), lambda i: (0, 0)),
        ],
        out_specs=pl.BlockSpec((1, TM), lambda i: (0, i)),
        out_shape=jax.ShapeDtypeStruct((1, M), jnp.int32),
        compiler_params=pltpu.CompilerParams(
            dimension_semantics=("parallel",)),
    )(x, cb, xsq, csq, iota_row)


def _sc_stage(cb, idx, xf, xd1):
    stage2 = xd1 is not None
    info = plsc.get_sparse_core_info()
    NC, NS, L = info.num_cores, info.num_subcores, info.num_lanes
    NW = NC * NS
    M = idx.shape[0]
    bpw = M // NW
    zsl = NB // NS
    CPT = D // L
    mesh = plsc.VectorSubcoreMesh(core_axis_name="c", subcore_axis_name="s")

    scratch = [
        pltpu.VMEM((bpw,), jnp.int32),
        pltpu.VMEM((bpw, D), jnp.float32),
        pltpu.VMEM((bpw, D), jnp.float32),
        pltpu.VMEM((bpw,), jnp.float32),
        pltpu.VMEM((zsl,), jnp.float32),
        pltpu.VMEM_SHARED((NB,), jnp.float32),
        pltpu.SemaphoreType.DMA,
    ]
    if stage2:
        scratch.insert(3, pltpu.VMEM((bpw, D), jnp.float32))

    @functools.partial(
        pl.kernel,
        out_type=[jax.ShapeDtypeStruct((M, D), jnp.float32),
                  jax.ShapeDtypeStruct((M, D), jnp.float32),
                  jax.ShapeDtypeStruct((NC, NB), jnp.float32)],
        mesh=mesh,
        scratch_types=scratch,
        compiler_params=pltpu.CompilerParams(use_tc_tiling_on_sc=False),
    )
    def k(cb_hbm, idx_hbm, xf_hbm, *rest):
        if stage2:
            (xd1_hbm, rows_hbm, ew_hbm, cnt_hbm,
             idx_v, rows_v, xf_v, xd1_v, ones_v, z_v, cnt_sp, sem) = rest
        else:
            (rows_hbm, ew_hbm, cnt_hbm,
             idx_v, rows_v, xf_v, ones_v, z_v, cnt_sp, sem) = rest
        c = lax.axis_index("c")
        s = lax.axis_index("s")
        wid = s * NC + c
        base = wid * bpw

        def zbody(i, _):
            z_v[pl.ds(i * L, L)] = jnp.zeros((L,), jnp.float32)
            return 0

        lax.fori_loop(0, zsl // L, zbody, 0, unroll=True)
        pltpu.sync_copy(z_v, cnt_sp.at[pl.ds(s * zsl, zsl)])

        def obody(i, _):
            ones_v[pl.ds(i * L, L)] = jnp.full((L,), 1.0, jnp.float32)
            return 0

        lax.fori_loop(0, bpw // L, obody, 0, unroll=True)

        pltpu.sync_copy(idx_hbm.at[pl.ds(base, bpw)], idx_v)
        pltpu.async_copy(cb_hbm.at[idx_v], rows_v, sem).wait()
        pltpu.sync_copy(rows_v, rows_hbm.at[pl.ds(base, bpw)])

        pltpu.sync_copy(xf_hbm.at[pl.ds(base, bpw)], xf_v)
        if stage2:
            pltpu.sync_copy(xd1_hbm.at[pl.ds(base, bpw)], xd1_v)

            def ebody(t, _):
                for cc in range(CPT):
                    sl = pl.ds(cc * L, L)
                    xfv = xf_v[t, sl]
                    rows_v[t, sl] = xfv + ((xd1_v[t, sl] + rows_v[t, sl])
                                           - xfv)
                return 0
        else:

            def ebody(t, _):
                for cc in range(CPT):
                    sl = pl.ds(cc * L, L)
                    rows_v[t, sl] = xf_v[t, sl] - rows_v[t, sl]
                return 0

        lax.fori_loop(0, bpw, ebody, 0)
        pltpu.sync_copy(rows_v, ew_hbm.at[pl.ds(base, bpw)])

        plsc.subcore_barrier()
        pltpu.sync_copy(ones_v, cnt_sp.at[idx_v], add=True)
        plsc.subcore_barrier()

        @pl.when(s == 0)
        def _():
            pltpu.sync_copy(cnt_sp, cnt_hbm.at[c])

    if stage2:
        return k(cb, idx, xf, xd1)
    return k(cb, idx, xf)


def _perplexity_from_counts(cnt):
    code_count = cnt[0] + cnt[1]
    prob = code_count / jnp.sum(code_count)
    return jnp.exp(-jnp.sum(prob * jnp.log(prob + 1e-07)))


def kernel(x, codebook1, codebook2):
    N, width, T = x.shape
    xf = jnp.transpose(x, (0, 2, 1)).reshape(-1, width)

    xsq1 = jnp.sum(xf ** 2, axis=-1, keepdims=True)
    csq1 = jnp.sum(codebook1 ** 2, axis=-1)[None, :]
    idx1 = _argmin_call(xf, codebook1, xsq1, csq1).reshape(-1)
    x_d1, x_res, cnt1 = _sc_stage(codebook1, idx1, xf, None)

    xsq2 = jnp.sum(x_res ** 2, axis=-1, keepdims=True)
    csq2 = jnp.sum(codebook2 ** 2, axis=-1)[None, :]
    idx2 = _argmin_call(x_res, codebook2, xsq2, csq2).reshape(-1)
    x_d2, x_d_flat, cnt2 = _sc_stage(codebook2, idx2, xf, x_d1)

    perplexity1 = _perplexity_from_counts(cnt1)
    perplexity2 = _perplexity_from_counts(cnt2)

    x_d = jnp.transpose(x_d_flat.reshape(N, T, width), (0, 2, 1))
    return (x_d, (xf, x_d1, x_d2), (perplexity1, perplexity2))


# speedup vs baseline: 1.0094x; 1.0094x over previous
import functools

import jax
import jax.numpy as jnp
from jax import lax
from jax.experimental import pallas as pl
from jax.experimental.pallas import tpu as pltpu
from jax.experimental.pallas import tpu_sc as plsc

NB = 8192
D = 64
TM = 1024

def _argmin_body(x_ref, cb_ref, xsq_ref, csq_ref, iota_ref, idx_ref):
    xs = x_ref[...] * -2.0
    mm = lax.dot_general(xs, cb_ref[...], (((1,), (1,)), ((), ())),
                         preferred_element_type=jnp.float32)
    dist = (xsq_ref[...] + mm) + csq_ref[...]
    m = jnp.min(dist, axis=-1, keepdims=True)
    cand = jnp.where(dist == m, iota_ref[...], jnp.float32(NB))
    col = jnp.min(cand, axis=-1, keepdims=True).astype(jnp.int32)
    idx_ref[...] = lax.transpose(col, (1, 0))


def _argmin_call(x, cb, xsq, csq):
    M = x.shape[0]
    iota_row = jnp.arange(NB, dtype=jnp.float32)[None, :]
    return pl.pallas_call(
        _argmin_body,
        grid=(M // TM,),
        in_specs=[
            pl.BlockSpec((TM, D), lambda i: (i, 0)),
            pl.BlockSpec((NB, D), lambda i: (0, 0)),
            pl.BlockSpec((TM, 1), lambda i: (i, 0)),
            pl.BlockSpec((1, NB), lambda i: (0, 0)),
            pl.BlockSpec((1, NB), lambda i: (0, 0)),
        ],
        out_specs=pl.BlockSpec((1, TM), lambda i: (0, i)),
        out_shape=jax.ShapeDtypeStruct((1, M), jnp.int32),
        compiler_params=pltpu.CompilerParams(
            dimension_semantics=("parallel",)),
    )(x, cb, xsq, csq, iota_row)


def _sc_stage(cb, idx, xf, xd1):
    stage2 = xd1 is not None
    info = plsc.get_sparse_core_info()
    NC, NS, L = info.num_cores, info.num_subcores, info.num_lanes
    NW = NC * NS
    M = idx.shape[0]
    bpw = M // NW
    zsl = NB // NS
    CPT = D // L
    mesh = plsc.VectorSubcoreMesh(core_axis_name="c", subcore_axis_name="s")

    scratch = [
        pltpu.VMEM((bpw,), jnp.int32),
        pltpu.VMEM((bpw, D), jnp.float32),
        pltpu.VMEM((bpw, D), jnp.float32),
        pltpu.SemaphoreType.DMA,
    ]
    if stage2:
        scratch.insert(3, pltpu.VMEM((bpw, D), jnp.float32))

    @functools.partial(
        pl.kernel,
        out_type=[jax.ShapeDtypeStruct((M, D), jnp.float32),
                  jax.ShapeDtypeStruct((M, D), jnp.float32)],
        mesh=mesh,
        scratch_types=scratch,
        compiler_params=pltpu.CompilerParams(use_tc_tiling_on_sc=False),
    )
    def k(cb_hbm, idx_hbm, xf_hbm, *rest):
        if stage2:
            (xd1_hbm, rows_hbm, ew_hbm,
             idx_v, rows_v, xf_v, xd1_v, sem) = rest
        else:
            (rows_hbm, ew_hbm,
             idx_v, rows_v, xf_v, sem) = rest
        c = lax.axis_index("c")
        s = lax.axis_index("s")
        wid = s * NC + c
        base = wid * bpw

        pltpu.sync_copy(idx_hbm.at[pl.ds(base, bpw)], idx_v)
        pltpu.async_copy(cb_hbm.at[idx_v], rows_v, sem).wait()
        pltpu.sync_copy(rows_v, rows_hbm.at[pl.ds(base, bpw)])

        pltpu.sync_copy(xf_hbm.at[pl.ds(base, bpw)], xf_v)
        if stage2:
            pltpu.sync_copy(xd1_hbm.at[pl.ds(base, bpw)], xd1_v)

            def ebody(t, _):
                for cc in range(CPT):
                    sl = pl.ds(cc * L, L)
                    xfv = xf_v[t, sl]
                    rows_v[t, sl] = xfv + ((xd1_v[t, sl] + rows_v[t, sl])
                                           - xfv)
                return 0
        else:

            def ebody(t, _):
                for cc in range(CPT):
                    sl = pl.ds(cc * L, L)
                    rows_v[t, sl] = xf_v[t, sl] - rows_v[t, sl]
                return 0

        lax.fori_loop(0, bpw, ebody, 0)
        pltpu.sync_copy(rows_v, ew_hbm.at[pl.ds(base, bpw)])

    if stage2:
        return k(cb, idx, xf, xd1)
    return k(cb, idx, xf)


def _sc_hist(idx):
    info = plsc.get_sparse_core_info()
    NC, NS, L = info.num_cores, info.num_subcores, info.num_lanes
    NW = NC * NS
    M = idx.shape[0]
    bpw = M // NW
    zsl = NB // NS
    mesh = plsc.VectorSubcoreMesh(core_axis_name="c", subcore_axis_name="s")

    @functools.partial(
        pl.kernel,
        out_type=jax.ShapeDtypeStruct((NC, NB), jnp.float32),
        mesh=mesh,
        scratch_types=[
            pltpu.VMEM((bpw,), jnp.int32),
            pltpu.VMEM((bpw,), jnp.float32),
            pltpu.VMEM((zsl,), jnp.float32),
            pltpu.VMEM_SHARED((NB,), jnp.float32),
        ],
        compiler_params=pltpu.CompilerParams(use_tc_tiling_on_sc=False),
    )
    def k(idx_hbm, cnt_hbm, idx_v, ones_v, z_v, cnt_sp):
        c = lax.axis_index("c")
        s = lax.axis_index("s")
        wid = s * NC + c
        base = wid * bpw

        def zbody(i, _):
            z_v[pl.ds(i * L, L)] = jnp.zeros((L,), jnp.float32)
            return 0

        lax.fori_loop(0, zsl // L, zbody, 0, unroll=True)
        pltpu.sync_copy(z_v, cnt_sp.at[pl.ds(s * zsl, zsl)])

        def obody(i, _):
            ones_v[pl.ds(i * L, L)] = jnp.full((L,), 1.0, jnp.float32)
            return 0

        lax.fori_loop(0, bpw // L, obody, 0, unroll=True)

        pltpu.sync_copy(idx_hbm.at[pl.ds(base, bpw)], idx_v)

        plsc.subcore_barrier()
        pltpu.sync_copy(ones_v, cnt_sp.at[idx_v], add=True)
        plsc.subcore_barrier()

        @pl.when(s == 0)
        def _():
            pltpu.sync_copy(cnt_sp, cnt_hbm.at[c])

    return k(idx)


def _perplexity_from_counts(cnt):
    code_count = cnt[0] + cnt[1]
    prob = code_count / jnp.sum(code_count)
    return jnp.exp(-jnp.sum(prob * jnp.log(prob + 1e-07)))


def kernel(x, codebook1, codebook2):
    N, width, T = x.shape
    xf = jnp.transpose(x, (0, 2, 1)).reshape(-1, width)

    xsq1 = jnp.sum(xf ** 2, axis=-1, keepdims=True)
    csq1 = jnp.sum(codebook1 ** 2, axis=-1)[None, :]
    idx1 = _argmin_call(xf, codebook1, xsq1, csq1).reshape(-1)
    x_d1, x_res = _sc_stage(codebook1, idx1, xf, None)
    cnt1 = _sc_hist(idx1)

    xsq2 = jnp.sum(x_res ** 2, axis=-1, keepdims=True)
    csq2 = jnp.sum(codebook2 ** 2, axis=-1)[None, :]
    idx2 = _argmin_call(x_res, codebook2, xsq2, csq2).reshape(-1)
    x_d2, x_d_flat = _sc_stage(codebook2, idx2, xf, x_d1)
    cnt2 = _sc_hist(idx2)

    perplexity1 = _perplexity_from_counts(cnt1)
    perplexity2 = _perplexity_from_counts(cnt2)

    x_d = jnp.transpose(x_d_flat.reshape(N, T, width), (0, 2, 1))
    return (x_d, (xf, x_d1, x_d2), (perplexity1, perplexity2))


# speedup vs baseline: 1.0113x; 1.0018x over previous
import functools

import jax
import jax.numpy as jnp
from jax import lax
from jax.experimental import pallas as pl
from jax.experimental.pallas import tpu as pltpu
from jax.experimental.pallas import tpu_sc as plsc

NB = 8192
D = 64
TM = 1024

def _argmin_body(x_ref, cb_ref, xsq_ref, csq_ref, iota_ref, idx_ref):
    xs = x_ref[...] * -2.0
    mm = lax.dot_general(xs, cb_ref[...], (((1,), (1,)), ((), ())),
                         preferred_element_type=jnp.float32)
    dist = (xsq_ref[...] + mm) + csq_ref[...]
    m = jnp.min(dist, axis=-1, keepdims=True)
    cand = jnp.where(dist == m, iota_ref[...], jnp.float32(NB))
    col = jnp.min(cand, axis=-1, keepdims=True).astype(jnp.int32)
    idx_ref[...] = lax.transpose(col, (1, 0))


def _argmin_call(x, cb, xsq, csq):
    M = x.shape[0]
    iota_row = jnp.arange(NB, dtype=jnp.float32)[None, :]
    return pl.pallas_call(
        _argmin_body,
        grid=(M // TM,),
        in_specs=[
            pl.BlockSpec((TM, D), lambda i: (i, 0)),
            pl.BlockSpec((NB, D), lambda i: (0, 0)),
            pl.BlockSpec((TM, 1), lambda i: (i, 0)),
            pl.BlockSpec((1, NB), lambda i: (0, 0)),
            pl.BlockSpec((1, NB), lambda i: (0, 0)),
        ],
        out_specs=pl.BlockSpec((1, TM), lambda i: (0, i)),
        out_shape=jax.ShapeDtypeStruct((1, M), jnp.int32),
        compiler_params=pltpu.CompilerParams(
            dimension_semantics=("parallel",)),
    )(x, cb, xsq, csq, iota_row)


def _sc_stage(cb, idx, xf, xd1):
    stage2 = xd1 is not None
    info = plsc.get_sparse_core_info()
    NC, NS, L = info.num_cores, info.num_subcores, info.num_lanes
    NW = NC * NS
    M = idx.shape[0]
    bpw = M // NW
    CPT = D // L
    mesh = plsc.VectorSubcoreMesh(core_axis_name="c", subcore_axis_name="s")

    scratch = [
        pltpu.VMEM((bpw,), jnp.int32),
        pltpu.VMEM((bpw, D), jnp.float32),
        pltpu.VMEM((bpw, D), jnp.float32),
        pltpu.SemaphoreType.DMA,
    ]
    if stage2:
        scratch.insert(3, pltpu.VMEM((bpw, D), jnp.float32))

    @functools.partial(
        pl.kernel,
        out_type=[jax.ShapeDtypeStruct((M, D), jnp.float32),
                  jax.ShapeDtypeStruct((M, D), jnp.float32)],
        mesh=mesh,
        scratch_types=scratch,
        compiler_params=pltpu.CompilerParams(use_tc_tiling_on_sc=False),
    )
    def k(cb_hbm, idx_hbm, xf_hbm, *rest):
        if stage2:
            (xd1_hbm, rows_hbm, ew_hbm,
             idx_v, rows_v, xf_v, xd1_v, sem) = rest
        else:
            (rows_hbm, ew_hbm,
             idx_v, rows_v, xf_v, sem) = rest
        c = lax.axis_index("c")
        s = lax.axis_index("s")
        wid = s * NC + c
        base = wid * bpw

        pltpu.sync_copy(idx_hbm.at[pl.ds(base, bpw)], idx_v)
        pltpu.async_copy(cb_hbm.at[idx_v], rows_v, sem).wait()
        pltpu.sync_copy(rows_v, rows_hbm.at[pl.ds(base, bpw)])

        pltpu.sync_copy(xf_hbm.at[pl.ds(base, bpw)], xf_v)
        if stage2:
            pltpu.sync_copy(xd1_hbm.at[pl.ds(base, bpw)], xd1_v)

            def ebody(t, _):
                for cc in range(CPT):
                    sl = pl.ds(cc * L, L)
                    xfv = xf_v[t, sl]
                    rows_v[t, sl] = xfv + ((xd1_v[t, sl] + rows_v[t, sl])
                                           - xfv)
                return 0
        else:

            def ebody(t, _):
                for cc in range(CPT):
                    sl = pl.ds(cc * L, L)
                    rows_v[t, sl] = xf_v[t, sl] - rows_v[t, sl]
                return 0

        lax.fori_loop(0, bpw, ebody, 0)
        pltpu.sync_copy(rows_v, ew_hbm.at[pl.ds(base, bpw)])

    if stage2:
        return k(cb, idx, xf, xd1)
    return k(cb, idx, xf)


def _sc_hist(idx):
    info = plsc.get_sparse_core_info()
    NC, NS, L = info.num_cores, info.num_subcores, info.num_lanes
    NW = NC * NS
    M = idx.shape[0]
    bpw = M // NW
    zsl = NB // NS
    mesh = plsc.VectorSubcoreMesh(core_axis_name="c", subcore_axis_name="s")

    @functools.partial(
        pl.kernel,
        out_type=jax.ShapeDtypeStruct((NC, NB), jnp.float32),
        mesh=mesh,
        scratch_types=[
            pltpu.VMEM((bpw,), jnp.int32),
            pltpu.VMEM((bpw,), jnp.float32),
            pltpu.VMEM((zsl,), jnp.float32),
            pltpu.VMEM_SHARED((NB,), jnp.float32),
        ],
        compiler_params=pltpu.CompilerParams(use_tc_tiling_on_sc=False),
    )
    def k(idx_hbm, cnt_hbm, idx_v, ones_v, z_v, cnt_sp):
        c = lax.axis_index("c")
        s = lax.axis_index("s")
        wid = s * NC + c
        base = wid * bpw

        def zbody(i, _):
            z_v[pl.ds(i * L, L)] = jnp.zeros((L,), jnp.float32)
            return 0

        lax.fori_loop(0, zsl // L, zbody, 0, unroll=True)
        pltpu.sync_copy(z_v, cnt_sp.at[pl.ds(s * zsl, zsl)])

        def obody(i, _):
            ones_v[pl.ds(i * L, L)] = jnp.full((L,), 1.0, jnp.float32)
            return 0

        lax.fori_loop(0, bpw // L, obody, 0, unroll=True)

        pltpu.sync_copy(idx_hbm.at[pl.ds(base, bpw)], idx_v)

        plsc.subcore_barrier()
        pltpu.sync_copy(ones_v, cnt_sp.at[idx_v], add=True)
        plsc.subcore_barrier()

        @pl.when(s == 0)
        def _():
            pltpu.sync_copy(cnt_sp, cnt_hbm.at[c])

    return k(idx)


def _perplexity_from_counts(cnt):
    code_count = cnt[0] + cnt[1]
    prob = code_count / jnp.sum(code_count)
    return jnp.exp(-jnp.sum(prob * jnp.log(prob + 1e-07)))


def kernel(x, codebook1, codebook2):
    N, width, T = x.shape
    xf = jnp.transpose(x, (0, 2, 1)).reshape(-1, width)

    xsq1 = jnp.sum(xf ** 2, axis=-1, keepdims=True)
    csq1 = jnp.sum(codebook1 ** 2, axis=-1)[None, :]
    idx1 = _argmin_call(xf, codebook1, xsq1, csq1).reshape(-1)
    x_d1, x_res = _sc_stage(codebook1, idx1, xf, None)
    cnt1 = _sc_hist(idx1)

    xsq2 = jnp.sum(x_res ** 2, axis=-1, keepdims=True)
    csq2 = jnp.sum(codebook2 ** 2, axis=-1)[None, :]
    idx2 = _argmin_call(x_res, codebook2, xsq2, csq2).reshape(-1)
    x_d2, x_d_flat = _sc_stage(codebook2, idx2, xf, x_d1)
    cnt2 = _sc_hist(idx2)

    perplexity1 = _perplexity_from_counts(cnt1)
    perplexity2 = _perplexity_from_counts(cnt2)

    x_d = jnp.transpose(x_d_flat.reshape(N, T, width), (0, 2, 1))
    return (x_d, (xf, x_d1, x_d2), (perplexity1, perplexity2))
